# baseline (speedup 1.0000x reference)
import functools

import jax
import jax.numpy as jnp
from jax.experimental import pallas as pl
from jax.experimental.pallas import tpu as pltpu

_BM = 512
_K = 4
_NC = 4


def _gcn_body(w_ref, bias_ref, uf_hbm, vf_hbm, sup_hbm, supt_hbm,
              out_u_ref, out_v_ref,
              uf_s, vf_s, sup_buf, supt_buf, tmpu_ref, tmpv_ref,
              sem_aux, sem_u, sem_v,
              *, n_r, nu, nv, n_i):
    i = pl.program_id(0)
    r = pl.program_id(1)
    s = i * n_r + r
    n_s = n_i * n_r

    def start(step, slot):
        if isinstance(step, int):
            step = jnp.int32(step)
        if isinstance(slot, int):
            slot = jnp.int32(slot)
        ii = step // n_r
        rr = step % n_r
        cw = nv // _NC
        for c in range(_NC):
            pltpu.make_async_copy(
                sup_hbm.at[pl.ds(ii * _BM, _BM), rr, pl.ds(c * cw, cw)],
                sup_buf.at[slot, :, pl.ds(c * cw, cw)],
                sem_u.at[slot, c]).start()
        for c in range(_NC):
            pltpu.make_async_copy(
                supt_hbm.at[pl.ds(ii * _BM, _BM), rr, pl.ds(c * cw, cw)],
                supt_buf.at[slot, :, pl.ds(c * cw, cw)],
                sem_v.at[slot, c]).start(priority=1)

    @pl.when(s == 0)
    def _prologue_copies():
        pltpu.make_async_copy(
            uf_hbm.at[pl.ds(0, nu), :], uf_s, sem_aux.at[0]).start()
        pltpu.make_async_copy(
            vf_hbm.at[pl.ds(0, nv), :], vf_s, sem_aux.at[1]).start()
        for t in range(_K - 1):
            if t < n_i * n_r:
                start(t, t % _K)

    @pl.when(s + _K - 1 < n_s)
    def _next_copy():
        start(s + _K - 1, (s + _K - 1) % _K)

    @pl.when(s == 0)
    def _build_tmps():
        pltpu.make_async_copy(
            uf_hbm.at[pl.ds(0, nu), :], uf_s, sem_aux.at[0]).wait()
        pltpu.make_async_copy(
            vf_hbm.at[pl.ds(0, nv), :], vf_s, sem_aux.at[1]).wait()
        wc = w_ref[0]
        for rr in range(n_r):
            if rr:
                wc = wc + w_ref[rr]
            tmpu_ref[rr * nu:(rr + 1) * nu, :] = jnp.dot(
                uf_s[...], wc, preferred_element_type=jnp.float32)
            tmpv_ref[rr * nv:(rr + 1) * nv, :] = jnp.dot(
                vf_s[...], wc, preferred_element_type=jnp.float32)

    slot = s % _K
    cw = nv // _NC
    for c in range(_NC):
        pltpu.make_async_copy(
            sup_hbm.at[pl.ds(i * _BM, _BM), r, pl.ds(c * cw, cw)],
            sup_buf.at[slot, :, pl.ds(c * cw, cw)],
            sem_u.at[slot, c]).wait()
    for c in range(_NC):
        pltpu.make_async_copy(
            supt_hbm.at[pl.ds(i * _BM, _BM), r, pl.ds(c * cw, cw)],
            supt_buf.at[slot, :, pl.ds(c * cw, cw)],
            sem_v.at[slot, c]).wait()

    part_u = jnp.dot(sup_buf[slot], tmpv_ref[pl.ds(r * nv, nv), :],
                     preferred_element_type=jnp.float32)
    part_v = jnp.dot(supt_buf[slot], tmpu_ref[pl.ds(r * nu, nu), :],
                     preferred_element_type=jnp.float32)

    @pl.when(r == 0)
    def _init():
        out_u_ref[...] = part_u
        out_v_ref[...] = part_v

    @pl.when((r > 0) & (r < n_r - 1))
    def _acc():
        out_u_ref[...] += part_u
        out_v_ref[...] += part_v

    @pl.when(r == n_r - 1)
    def _finish():
        bias = bias_ref[...]
        out_u_ref[...] = jnp.maximum(out_u_ref[...] + part_u + bias, 0.0)
        out_v_ref[...] = jnp.maximum(out_v_ref[...] + part_v + bias, 0.0)


def kernel(u_feat, v_feat, support, support_t, u_weight, u_bias):
    nu, d = u_feat.shape
    nv = v_feat.shape[0]
    n_r = support.shape[1]
    h = u_weight.shape[2]
    n_i = nu // _BM

    bias2 = u_bias.reshape(1, h)

    grid = (n_i, n_r)

    out_u, out_v = pl.pallas_call(
        functools.partial(_gcn_body, n_r=n_r, nu=nu, nv=nv, n_i=n_i),
        grid=grid,
        in_specs=[
            pl.BlockSpec((n_r, d, h), lambda i, r: (0, 0, 0)),
            pl.BlockSpec((1, h), lambda i, r: (0, 0)),
            pl.BlockSpec(memory_space=pltpu.MemorySpace.HBM),
            pl.BlockSpec(memory_space=pltpu.MemorySpace.HBM),
            pl.BlockSpec(memory_space=pltpu.MemorySpace.HBM),
            pl.BlockSpec(memory_space=pltpu.MemorySpace.HBM),
        ],
        out_specs=[
            pl.BlockSpec((_BM, h), lambda i, r: (i, 0)),
            pl.BlockSpec((_BM, h), lambda i, r: (i, 0)),
        ],
        out_shape=[
            jax.ShapeDtypeStruct((nu, h), jnp.float32),
            jax.ShapeDtypeStruct((nv, h), jnp.float32),
        ],
        scratch_shapes=[
            pltpu.VMEM((nu, d), jnp.float32),
            pltpu.VMEM((nv, d), jnp.float32),
            pltpu.VMEM((_K, _BM, nv), jnp.float32),
            pltpu.VMEM((_K, _BM, nu), jnp.float32),
            pltpu.VMEM((n_r * nu, h), jnp.float32),
            pltpu.VMEM((n_r * nv, h), jnp.float32),
            pltpu.SemaphoreType.DMA((2,)),
            pltpu.SemaphoreType.DMA((_K, _NC)),
            pltpu.SemaphoreType.DMA((_K, _NC)),
        ],
    )(u_weight, bias2, u_feat, v_feat, support, support_t)

    return (out_u, out_v)


# speedup vs baseline: 1.0066x; 1.0066x over previous
import functools

import jax
import jax.numpy as jnp
from jax.experimental import pallas as pl
from jax.experimental.pallas import tpu as pltpu

_BM = 256
_K = 4
_NC = 4


def _gcn_body(w_ref, uf_ref, vf_ref, bias_ref, sup_hbm, supt_hbm,
              out_u_ref, out_v_ref,
              sup_buf, supt_buf, tmpu_ref, tmpv_ref, sem_u, sem_v,
              *, n_r, nu, nv, n_i):
    i = pl.program_id(0)
    r = pl.program_id(1)
    s = i * n_r + r
    n_s = n_i * n_r

    def start(step, slot):
        if isinstance(step, int):
            step = jnp.int32(step)
        if isinstance(slot, int):
            slot = jnp.int32(slot)
        ii = step // n_r
        rr = step % n_r
        cw = nv // _NC
        for c in range(_NC):
            pltpu.make_async_copy(
                sup_hbm.at[pl.ds(ii * _BM, _BM), rr, pl.ds(c * cw, cw)],
                sup_buf.at[slot, :, pl.ds(c * cw, cw)],
                sem_u.at[slot, c]).start()
        for c in range(_NC):
            pltpu.make_async_copy(
                supt_hbm.at[pl.ds(ii * _BM, _BM), rr, pl.ds(c * cw, cw)],
                supt_buf.at[slot, :, pl.ds(c * cw, cw)],
                sem_v.at[slot, c]).start()

    @pl.when(s == 0)
    def _prologue_copies():
        for t in range(_K - 1):
            if t < n_i * n_r:
                start(t, t % _K)

    @pl.when(s + _K - 1 < n_s)
    def _next_copy():
        start(s + _K - 1, (s + _K - 1) % _K)

    @pl.when(s == 0)
    def _build_tmps():
        wc = w_ref[0]
        for rr in range(n_r):
            if rr:
                wc = wc + w_ref[rr]
            tmpu_ref[rr * nu:(rr + 1) * nu, :] = jnp.dot(
                uf_ref[...], wc, preferred_element_type=jnp.float32)
            tmpv_ref[rr * nv:(rr + 1) * nv, :] = jnp.dot(
                vf_ref[...], wc, preferred_element_type=jnp.float32)

    slot = s % _K
    cw = nv // _NC
    for c in range(_NC):
        pltpu.make_async_copy(
            sup_hbm.at[pl.ds(i * _BM, _BM), r, pl.ds(c * cw, cw)],
            sup_buf.at[slot, :, pl.ds(c * cw, cw)],
            sem_u.at[slot, c]).wait()
    for c in range(_NC):
        pltpu.make_async_copy(
            supt_hbm.at[pl.ds(i * _BM, _BM), r, pl.ds(c * cw, cw)],
            supt_buf.at[slot, :, pl.ds(c * cw, cw)],
            sem_v.at[slot, c]).wait()

    part_u = jnp.dot(sup_buf[slot], tmpv_ref[pl.ds(r * nv, nv), :],
                     preferred_element_type=jnp.float32)
    part_v = jnp.dot(supt_buf[slot], tmpu_ref[pl.ds(r * nu, nu), :],
                     preferred_element_type=jnp.float32)

    @pl.when(r == 0)
    def _init():
        out_u_ref[...] = part_u
        out_v_ref[...] = part_v

    @pl.when((r > 0) & (r < n_r - 1))
    def _acc():
        out_u_ref[...] += part_u
        out_v_ref[...] += part_v

    @pl.when(r == n_r - 1)
    def _finish():
        bias = bias_ref[...]
        out_u_ref[...] = jnp.maximum(out_u_ref[...] + part_u + bias, 0.0)
        out_v_ref[...] = jnp.maximum(out_v_ref[...] + part_v + bias, 0.0)


def kernel(u_feat, v_feat, support, support_t, u_weight, u_bias):
    nu, d = u_feat.shape
    nv = v_feat.shape[0]
    n_r = support.shape[1]
    h = u_weight.shape[2]
    n_i = nu // _BM

    bias2 = u_bias.reshape(1, h)

    grid = (n_i, n_r)

    out_u, out_v = pl.pallas_call(
        functools.partial(_gcn_body, n_r=n_r, nu=nu, nv=nv, n_i=n_i),
        grid=grid,
        in_specs=[
            pl.BlockSpec((n_r, d, h), lambda i, r: (0, 0, 0)),
            pl.BlockSpec((nu, d), lambda i, r: (0, 0)),
            pl.BlockSpec((nv, d), lambda i, r: (0, 0)),
            pl.BlockSpec((1, h), lambda i, r: (0, 0)),
            pl.BlockSpec(memory_space=pltpu.MemorySpace.HBM),
            pl.BlockSpec(memory_space=pltpu.MemorySpace.HBM),
        ],
        out_specs=[
            pl.BlockSpec((_BM, h), lambda i, r: (i, 0)),
            pl.BlockSpec((_BM, h), lambda i, r: (i, 0)),
        ],
        out_shape=[
            jax.ShapeDtypeStruct((nu, h), jnp.float32),
            jax.ShapeDtypeStruct((nv, h), jnp.float32),
        ],
        scratch_shapes=[
            pltpu.VMEM((_K, _BM, nv), jnp.float32),
            pltpu.VMEM((_K, _BM, nu), jnp.float32),
            pltpu.VMEM((n_r * nu, h), jnp.float32),
            pltpu.VMEM((n_r * nv, h), jnp.float32),
            pltpu.SemaphoreType.DMA((_K, _NC)),
            pltpu.SemaphoreType.DMA((_K, _NC)),
        ],
    )(u_weight, u_feat, v_feat, bias2, support, support_t)

    return (out_u, out_v)
